# Initial kernel scaffold; baseline (speedup 1.0000x reference)
import jax
import jax.numpy as jnp
from jax.experimental import pallas as pl


def kernel(x, codebook):
    raise NotImplementedError("write your pallas kernel here")


# speedup vs baseline: 1.9687x; 1.9687x over previous
import functools

import jax
import jax.numpy as jnp
from jax import lax
from jax.experimental import pallas as pl
from jax.experimental.pallas import tpu as pltpu
from jax.experimental.pallas import tpu_sc as plsc

TN = 512
TK = 2048


def _dist_argmin_kernel(nj, ni, n, lat, x_ref, cb_ref, idx_ref, loss_ref,
                        minv, mina):
    i = pl.program_id(0)
    j = pl.program_id(1)
    xb = x_ref[0]
    cb = cb_ref[...]
    cross = lax.dot_general(cb, xb, (((1,), (0,)), ((), ())),
                            preferred_element_type=jnp.float32)
    hc2 = 0.5 * jnp.sum(cb * cb, axis=1, keepdims=True)
    scores = hc2 - cross
    m = jnp.min(scores, axis=0)
    a = jnp.argmin(scores, axis=0).astype(jnp.int32) + j * TK

    @pl.when(j == 0)
    def _():
        minv[0, :] = m
        mina[0, :] = a

    @pl.when(j > 0)
    def _():
        better = m < minv[0, :]
        minv[0, :] = jnp.where(better, m, minv[0, :])
        mina[0, :] = jnp.where(better, a, mina[0, :])

    @pl.when(j == nj - 1)
    def _():
        idx_ref[0, 0, :] = mina[0, :]
        x2 = jnp.sum(xb * xb, axis=0)
        part = jnp.sum(x2 + 2.0 * minv[0, :])
        prev = jnp.where(i == 0, 0.0, loss_ref[0, 0])
        tot = prev + part
        loss_ref[0, 0] = jnp.where(i == ni - 1, tot / (n * lat), tot)


def _dist_argmin(x3, cb):
    b, lat, hw = x3.shape
    k = cb.shape[0]
    n = b * hw
    ni, nj = n // TN, k // TK
    tiles_per_b = hw // TN
    return pl.pallas_call(
        functools.partial(_dist_argmin_kernel, nj, ni, n, lat),
        grid=(ni, nj),
        in_specs=[
            pl.BlockSpec((1, lat, TN),
                         lambda i, j: (i // tiles_per_b, 0, i % tiles_per_b)),
            pl.BlockSpec((TK, lat), lambda i, j: (j, 0)),
        ],
        out_specs=[
            pl.BlockSpec((1, 1, TN), lambda i, j: (i, 0, 0)),
            pl.BlockSpec(memory_space=pltpu.SMEM),
        ],
        out_shape=[
            jax.ShapeDtypeStruct((ni, 1, TN), jnp.int32),
            jax.ShapeDtypeStruct((1, 1), jnp.float32),
        ],
        scratch_shapes=[
            pltpu.VMEM((1, TN), jnp.float32),
            pltpu.VMEM((1, TN), jnp.int32),
        ],
        compiler_params=pltpu.CompilerParams(
            dimension_semantics=("arbitrary", "arbitrary")),
    )(x3, cb)


def _gather_sc(cb, idx3):
    nw, nchunk, cw = idx3.shape
    n = nw * nchunk * cw
    bpw = n // nw
    lat = cb.shape[1]
    mesh = plsc.VectorSubcoreMesh(core_axis_name="c", subcore_axis_name="s")

    @functools.partial(
        pl.kernel, mesh=mesh,
        out_type=jax.ShapeDtypeStruct((n, lat), jnp.float32),
        scratch_types=[
            pltpu.VMEM((nchunk, cw), jnp.int32),
            pltpu.VMEM((bpw, lat), jnp.float32),
            pltpu.SemaphoreType.DMA,
        ],
    )
    def k(cb_hbm, idx_hbm, out_hbm, idx_v, rows_v, sem):
        wid = lax.axis_index("s") * 2 + lax.axis_index("c")
        pltpu.sync_copy(idx_hbm.at[wid], idx_v)
        for c in range(nchunk):
            pltpu.async_copy(cb_hbm.at[idx_v.at[c]],
                             rows_v.at[pl.ds(c * cw, cw)], sem).wait()
        pltpu.sync_copy(rows_v, out_hbm.at[pl.ds(wid * bpw, bpw)])

    return k(cb, idx3)


def kernel(x, codebook):
    b, lat, h, w = x.shape
    n = b * h * w
    x3 = x.reshape(b, lat, h * w)
    idx_blk, loss2 = _dist_argmin(x3, codebook)
    idx_flat = idx_blk.reshape(n)
    q = _gather_sc(codebook, idx_flat.reshape(32, n // 32 // 128, 128))
    out_q = q.reshape(b, h, w, lat).transpose(0, 3, 1, 2)
    loss = loss2[0, 0]
    return (out_q, idx_flat.reshape(b, h, w), loss, loss)


# speedup vs baseline: 2.5362x; 1.2883x over previous
import functools

import jax
import jax.numpy as jnp
from jax import lax
from jax.experimental import pallas as pl
from jax.experimental.pallas import tpu as pltpu
from jax.experimental.pallas import tpu_sc as plsc

TN = 512
TK = 2048


def _dist_argmin_kernel(nj, ni, n, lat, x_ref, cb_ref, idx_ref, loss_ref,
                        hc2_ref):
    i = pl.program_id(0)
    xb = x_ref[0]

    @pl.when(i == 0)
    def _():
        for jc in range(nj):
            cbt = cb_ref[jc * TK:(jc + 1) * TK, :]
            hc2_ref[jc * TK:(jc + 1) * TK, :] = 0.5 * jnp.sum(
                cbt * cbt, axis=1, keepdims=True)

    m = jnp.full((TN,), jnp.inf, jnp.float32)
    a = jnp.zeros((TN,), jnp.int32)
    for jc in range(nj):
        cbt = cb_ref[jc * TK:(jc + 1) * TK, :]
        cross = lax.dot_general(cbt, xb, (((1,), (0,)), ((), ())),
                                preferred_element_type=jnp.float32)
        scores = hc2_ref[jc * TK:(jc + 1) * TK, :] - cross
        ml = jnp.min(scores, axis=0)
        al = jnp.argmin(scores, axis=0).astype(jnp.int32) + jc * TK
        better = ml < m
        m = jnp.where(better, ml, m)
        a = jnp.where(better, al, a)

    idx_ref[0, 0, :] = a
    x2 = jnp.sum(xb * xb, axis=0)
    part = jnp.sum(x2 + 2.0 * m)
    prev = jnp.where(i == 0, 0.0, loss_ref[0, 0])
    tot = prev + part
    loss_ref[0, 0] = jnp.where(i == ni - 1, tot / (n * lat), tot)


def _dist_argmin(x3, cb):
    b, lat, hw = x3.shape
    k = cb.shape[0]
    n = b * hw
    ni, nj = n // TN, k // TK
    tiles_per_b = hw // TN
    return pl.pallas_call(
        functools.partial(_dist_argmin_kernel, nj, ni, n, lat),
        grid=(ni,),
        in_specs=[
            pl.BlockSpec((1, lat, TN),
                         lambda i: (i // tiles_per_b, 0, i % tiles_per_b)),
            pl.BlockSpec((k, lat), lambda i: (0, 0)),
        ],
        out_specs=[
            pl.BlockSpec((1, 1, TN), lambda i: (i, 0, 0)),
            pl.BlockSpec(memory_space=pltpu.SMEM),
        ],
        out_shape=[
            jax.ShapeDtypeStruct((ni, 1, TN), jnp.int32),
            jax.ShapeDtypeStruct((1, 1), jnp.float32),
        ],
        scratch_shapes=[
            pltpu.VMEM((k, 1), jnp.float32),
        ],
        compiler_params=pltpu.CompilerParams(
            dimension_semantics=("arbitrary",)),
    )(x3, cb)


def _gather_sc(cb, idx3):
    nw, nchunk, cw = idx3.shape
    n = nw * nchunk * cw
    bpw = n // nw
    lat = cb.shape[1]
    mesh = plsc.VectorSubcoreMesh(core_axis_name="c", subcore_axis_name="s")

    @functools.partial(
        pl.kernel, mesh=mesh,
        out_type=jax.ShapeDtypeStruct((n, lat), jnp.float32),
        scratch_types=[
            pltpu.VMEM((nchunk, cw), jnp.int32),
            pltpu.VMEM((bpw, lat), jnp.float32),
            pltpu.SemaphoreType.DMA,
        ],
    )
    def k(cb_hbm, idx_hbm, out_hbm, idx_v, rows_v, sem):
        wid = lax.axis_index("s") * 2 + lax.axis_index("c")
        pltpu.sync_copy(idx_hbm.at[wid], idx_v)
        for c in range(nchunk):
            pltpu.async_copy(cb_hbm.at[idx_v.at[c]],
                             rows_v.at[pl.ds(c * cw, cw)], sem).wait()
        pltpu.sync_copy(rows_v, out_hbm.at[pl.ds(wid * bpw, bpw)])

    return k(cb, idx3)


def kernel(x, codebook):
    b, lat, h, w = x.shape
    n = b * h * w
    x3 = x.reshape(b, lat, h * w)
    idx_blk, loss2 = _dist_argmin(x3, codebook)
    idx_flat = idx_blk.reshape(n)
    q = _gather_sc(codebook, idx_flat.reshape(32, n // 32 // 128, 128))
    out_q = q.reshape(b, h, w, lat).transpose(0, 3, 1, 2)
    loss = loss2[0, 0]
    return (out_q, idx_flat.reshape(b, h, w), loss, loss)
